# baseline (speedup 1.0000x reference)
import functools

import jax
import jax.numpy as jnp
from jax import lax
from jax.experimental import pallas as pl
from jax.experimental.pallas import tpu as pltpu


def _body(x_ref, q_ref, d_ref, y_ref, loss_ref, acc_ref, sm_ref, *, nb, d_dim):
    i = pl.program_id(0)
    x = x_ref[...]
    q = q_ref[...]
    diff = x - q
    sq = diff * diff
    ones = jnp.ones((1, x.shape[1]), jnp.float32)
    dr = lax.dot_general(ones, sq, (((1,), (1,)), ((), ())),
                         preferred_element_type=jnp.float32,
                         precision=lax.Precision.HIGHEST)
    d_ref[...] = dr

    logits = -dr
    mb = jnp.max(logits)

    @pl.when(i == 0)
    def _init():
        acc_ref[...] = jnp.zeros_like(acc_ref)
        sm_ref[0] = -jnp.inf
        sm_ref[1] = 0.0

    m_old = sm_ref[0]
    m_new = jnp.maximum(m_old, mb)
    c = jnp.exp(m_old - m_new)
    w = jnp.exp(logits - m_new)
    sm_ref[0] = m_new
    sm_ref[1] = sm_ref[1] * c + jnp.sum(w)
    acc_ref[...] = acc_ref[...] * c + lax.dot_general(
        w, x, (((1,), (0,)), ((), ())),
        preferred_element_type=jnp.float32,
        precision=lax.Precision.HIGHEST)

    @pl.when(i == nb - 1)
    def _fin():
        y = acc_ref[...] / sm_ref[1]
        y_ref[...] = y
        idx = lax.broadcasted_iota(jnp.int32, y.shape, 1)
        e = jnp.where(idx % 2 == 0, y - q, 0.0)
        loss_ref[...] = (jnp.sum(e * e) / d_dim).reshape(1, 1)


def kernel(input_mu, input_sig, on_states):
    k_dim, d_dim, _ = on_states.shape
    two_d = 2 * d_dim
    x = on_states.reshape(k_dim, two_d)
    q = jnp.stack([input_mu, input_sig], axis=-1).reshape(1, two_d)

    kb = 1024
    nb = k_dim // kb

    dists2d, y, loss = pl.pallas_call(
        functools.partial(_body, nb=nb, d_dim=d_dim),
        grid=(nb,),
        in_specs=[
            pl.BlockSpec((kb, two_d), lambda i: (i, 0)),
            pl.BlockSpec((1, two_d), lambda i: (0, 0)),
        ],
        out_specs=[
            pl.BlockSpec((1, kb), lambda i: (0, i)),
            pl.BlockSpec((1, two_d), lambda i: (0, 0)),
            pl.BlockSpec((1, 1), lambda i: (0, 0)),
        ],
        out_shape=[
            jax.ShapeDtypeStruct((1, k_dim), jnp.float32),
            jax.ShapeDtypeStruct((1, two_d), jnp.float32),
            jax.ShapeDtypeStruct((1, 1), jnp.float32),
        ],
        scratch_shapes=[
            pltpu.VMEM((1, two_d), jnp.float32),
            pltpu.SMEM((2,), jnp.float32),
        ],
    )(x, q)

    quantised = y[0, 0::2]
    loss_s = loss[0, 0]
    return (quantised, loss_s, loss_s, dists2d.reshape(k_dim))


# speedup vs baseline: 1.1773x; 1.1773x over previous
import functools

import jax
import jax.numpy as jnp
from jax import lax
from jax.experimental import pallas as pl
from jax.experimental.pallas import tpu as pltpu

_HIGH = lax.Precision.DEFAULT


def _body(mu_ref, sig_ref, x_ref, d_ref, y_ref, loss_ref,
          acc_ref, q_ref, sm_ref, *, nb, d_dim):
    i = pl.program_id(0)
    two_d = 2 * d_dim

    @pl.when(i == 0)
    def _init():
        lane = lax.broadcasted_iota(jnp.int32, (1, two_d), 1)
        mu2 = jnp.repeat(mu_ref[...], 2, axis=1)
        sig2 = jnp.repeat(sig_ref[...], 2, axis=1)
        q0 = jnp.where(lane % 2 == 0, mu2, sig2)
        q_ref[...] = q0
        acc_ref[...] = jnp.zeros_like(acc_ref)
        sm_ref[0] = -jnp.inf
        sm_ref[1] = 0.0
        sm_ref[2] = jnp.sum(q0 * q0)

    x = x_ref[...]
    q = q_ref[...]
    ones = jnp.ones((1, two_d), jnp.float32)
    ssum = lax.dot_general(ones, x * x, (((1,), (1,)), ((), ())),
                           preferred_element_type=jnp.float32,
                           precision=_HIGH)
    xq = lax.dot_general(q, x, (((1,), (1,)), ((), ())),
                         preferred_element_type=jnp.float32,
                         precision=_HIGH)
    d = ssum - 2.0 * xq + sm_ref[2]
    d_ref[...] = d

    logits = -d
    mb = jnp.max(logits)
    m_old = sm_ref[0]
    m_new = jnp.maximum(m_old, mb)
    c = jnp.exp(m_old - m_new)
    w = jnp.exp(logits - m_new)
    sm_ref[0] = m_new
    sm_ref[1] = sm_ref[1] * c + jnp.sum(w)
    acc_ref[...] = acc_ref[...] * c + lax.dot_general(
        w, x, (((1,), (0,)), ((), ())),
        preferred_element_type=jnp.float32, precision=_HIGH)

    @pl.when(i == nb - 1)
    def _fin():
        y = acc_ref[...] / sm_ref[1]
        row = lax.broadcasted_iota(jnp.int32, (two_d, d_dim), 0)
        col = lax.broadcasted_iota(jnp.int32, (two_d, d_dim), 1)
        esel = jnp.where(row == 2 * col, 1.0, 0.0)
        quant = lax.dot_general(y, esel, (((1,), (0,)), ((), ())),
                                preferred_element_type=jnp.float32,
                                precision=_HIGH)
        y_ref[...] = quant
        e = quant - mu_ref[...]
        loss_ref[...] = (jnp.sum(e * e) / d_dim).reshape(1, 1)


def kernel(input_mu, input_sig, on_states):
    k_dim, d_dim, _ = on_states.shape
    two_d = 2 * d_dim
    x = on_states.reshape(k_dim, two_d)
    mu = input_mu.reshape(1, d_dim)
    sig = input_sig.reshape(1, d_dim)

    kb = 1024
    nb = k_dim // kb

    dists2d, y, loss = pl.pallas_call(
        functools.partial(_body, nb=nb, d_dim=d_dim),
        grid=(nb,),
        in_specs=[
            pl.BlockSpec((1, d_dim), lambda i: (0, 0)),
            pl.BlockSpec((1, d_dim), lambda i: (0, 0)),
            pl.BlockSpec((kb, two_d), lambda i: (i, 0)),
        ],
        out_specs=[
            pl.BlockSpec((1, kb), lambda i: (0, i)),
            pl.BlockSpec((1, d_dim), lambda i: (0, 0)),
            pl.BlockSpec((1, 1), lambda i: (0, 0)),
        ],
        out_shape=[
            jax.ShapeDtypeStruct((1, k_dim), jnp.float32),
            jax.ShapeDtypeStruct((1, d_dim), jnp.float32),
            jax.ShapeDtypeStruct((1, 1), jnp.float32),
        ],
        scratch_shapes=[
            pltpu.VMEM((1, two_d), jnp.float32),
            pltpu.VMEM((1, two_d), jnp.float32),
            pltpu.SMEM((3,), jnp.float32),
        ],
    )(mu, sig, x)

    quantised = y.reshape(d_dim)
    loss_s = loss.reshape(())
    return (quantised, loss_s, loss_s, dists2d.reshape(k_dim))


# speedup vs baseline: 5.0929x; 4.3261x over previous
import functools

import jax
import jax.numpy as jnp
from jax import lax
from jax.experimental import pallas as pl
from jax.experimental.pallas import tpu as pltpu


def _body(mu_ref, sig_ref, x_ref, d_ref, y_ref, loss_ref,
          acc_ref, sm_ref, *, nb, d_dim):
    i = pl.program_id(0)

    mu = mu_ref[...]
    sig = sig_ref[...]

    @pl.when(i == 0)
    def _init():
        acc_ref[...] = jnp.zeros_like(acc_ref)
        sm_ref[0] = -jnp.inf
        sm_ref[1] = 0.0
        sm_ref[2] = jnp.sum(mu * mu) + jnp.sum(sig * sig)

    xm = x_ref[:, 0, :]
    xg = x_ref[:, 1, :]
    sq = xm * xm + xg * xg
    ones = jnp.ones((1, d_dim), jnp.float32)
    ssum = lax.dot_general(ones, sq, (((1,), (1,)), ((), ())),
                           preferred_element_type=jnp.float32)
    xqm = lax.dot_general(mu, xm, (((1,), (1,)), ((), ())),
                          preferred_element_type=jnp.float32)
    xqg = lax.dot_general(sig, xg, (((1,), (1,)), ((), ())),
                          preferred_element_type=jnp.float32)
    d = ssum - 2.0 * (xqm + xqg) + sm_ref[2]
    d_ref[...] = d

    logits = -d
    mb = jnp.max(logits)
    m_old = sm_ref[0]
    m_new = jnp.maximum(m_old, mb)
    c = jnp.exp(m_old - m_new)
    w = jnp.exp(logits - m_new)
    sm_ref[0] = m_new
    sm_ref[1] = sm_ref[1] * c + jnp.sum(w)
    acc_ref[...] = acc_ref[...] * c + lax.dot_general(
        w, xm, (((1,), (0,)), ((), ())),
        preferred_element_type=jnp.float32)

    @pl.when(i == nb - 1)
    def _fin():
        quant = acc_ref[...] / sm_ref[1]
        y_ref[...] = quant
        e = quant - mu
        loss_ref[...] = (jnp.sum(e * e) / d_dim).reshape(1, 1)


def kernel(input_mu, input_sig, on_states):
    k_dim, d_dim, _ = on_states.shape
    xs = on_states.transpose(0, 2, 1)
    mu = input_mu.reshape(1, d_dim)
    sig = input_sig.reshape(1, d_dim)

    kb = 1024
    nb = k_dim // kb

    dists2d, y, loss = pl.pallas_call(
        functools.partial(_body, nb=nb, d_dim=d_dim),
        grid=(nb,),
        in_specs=[
            pl.BlockSpec((1, d_dim), lambda i: (0, 0)),
            pl.BlockSpec((1, d_dim), lambda i: (0, 0)),
            pl.BlockSpec((kb, 2, d_dim), lambda i: (i, 0, 0)),
        ],
        out_specs=[
            pl.BlockSpec((1, kb), lambda i: (0, i)),
            pl.BlockSpec((1, d_dim), lambda i: (0, 0)),
            pl.BlockSpec((1, 1), lambda i: (0, 0)),
        ],
        out_shape=[
            jax.ShapeDtypeStruct((1, k_dim), jnp.float32),
            jax.ShapeDtypeStruct((1, d_dim), jnp.float32),
            jax.ShapeDtypeStruct((1, 1), jnp.float32),
        ],
        scratch_shapes=[
            pltpu.VMEM((1, d_dim), jnp.float32),
            pltpu.SMEM((3,), jnp.float32),
        ],
    )(mu, sig, xs)

    quantised = y.reshape(d_dim)
    loss_s = loss.reshape(())
    return (quantised, loss_s, loss_s, dists2d.reshape(k_dim))


# speedup vs baseline: 5.4295x; 1.0661x over previous
import functools

import jax
import jax.numpy as jnp
from jax import lax
from jax.experimental import pallas as pl
from jax.experimental.pallas import tpu as pltpu


def _roll(v, shift):
    n = v.shape[-1]
    return pltpu.roll(v, shift % n, axis=v.ndim - 1)


def _body(mu_ref, sig_ref, x_ref, d_ref, y_ref, loss_ref,
          qb_ref, mi_ref, dsp_ref, acc_ref, sm_ref,
          *, nb, d_dim, kb, s_rows):
    i = pl.program_id(0)
    half = d_dim // 2

    @pl.when(i == 0)
    def _init():
        mu = mu_ref[...]
        sig = sig_ref[...]
        rm = lax.broadcasted_iota(jnp.int32, (s_rows, 128), 0) % 4
        qb_ref[...] = jnp.where(
            rm == 0, jnp.broadcast_to(mu[:, :half], (s_rows, 128)),
            jnp.where(rm == 1, jnp.broadcast_to(sig[:, :half], (s_rows, 128)),
                      jnp.where(rm == 2,
                                jnp.broadcast_to(mu[:, half:], (s_rows, 128)),
                                jnp.broadcast_to(sig[:, half:],
                                                 (s_rows, 128)))))
        lane = lax.broadcasted_iota(jnp.int32, (1, s_rows), 1)
        mi_ref[...] = jnp.where(lane % 4 == 0, 0.0, jnp.inf)
        acc_ref[...] = jnp.zeros_like(acc_ref)
        sm_ref[0] = -jnp.inf
        sm_ref[1] = 0.0

    x = x_ref[...]
    diff = x - qb_ref[...]
    sq = diff * diff
    ones = jnp.ones((1, 128), jnp.float32)
    drow = lax.dot_general(ones, sq, (((1,), (1,)), ((), ())),
                           preferred_element_type=jnp.float32)
    dsp_ref[i, :] = drow[0, :]
    f = drow + _roll(drow, -1) + _roll(drow, -2) + _roll(drow, -3)
    fv = f + mi_ref[...]

    logits = -fv
    mb = jnp.max(logits)
    m_old = sm_ref[0]
    m_new = jnp.maximum(m_old, mb)
    c = jnp.exp(m_old - m_new)
    w = jnp.exp(logits - m_new)
    sm_ref[0] = m_new
    sm_ref[1] = sm_ref[1] * c + jnp.sum(w)
    w4 = jnp.concatenate(
        [w, _roll(w, 1), _roll(w, 2), _roll(w, 3)], axis=0)
    acc_ref[...] = acc_ref[...] * c + lax.dot_general(
        w4, x, (((1,), (0,)), ((), ())),
        preferred_element_type=jnp.float32)

    @pl.when(i == nb - 1)
    def _fin():
        dall = jnp.sum(dsp_ref[...].reshape(nb, kb, 4), axis=2)
        d_ref[...] = dall
        a = acc_ref[...] / sm_ref[1]
        quant = jnp.concatenate([a[0:1, :], a[2:3, :]], axis=1)
        y_ref[...] = quant
        e = quant - mu_ref[...]
        loss_ref[...] = (jnp.sum(e * e) / d_dim).reshape(1, 1)


def kernel(input_mu, input_sig, on_states):
    k_dim, d_dim, _ = on_states.shape
    x = (on_states.transpose(0, 2, 1)
         .reshape(k_dim, 2, 2, d_dim // 2)
         .transpose(0, 2, 1, 3)
         .reshape(4 * k_dim, d_dim // 2))
    mu = input_mu.reshape(1, d_dim)
    sig = input_sig.reshape(1, d_dim)

    kb = 1024
    nb = k_dim // kb
    s_rows = 4 * kb

    dists2d, y, loss = pl.pallas_call(
        functools.partial(_body, nb=nb, d_dim=d_dim, kb=kb, s_rows=s_rows),
        grid=(nb,),
        in_specs=[
            pl.BlockSpec((1, d_dim), lambda i: (0, 0)),
            pl.BlockSpec((1, d_dim), lambda i: (0, 0)),
            pl.BlockSpec((s_rows, 128), lambda i: (i, 0)),
        ],
        out_specs=[
            pl.BlockSpec((nb, kb), lambda i: (0, 0)),
            pl.BlockSpec((1, d_dim), lambda i: (0, 0)),
            pl.BlockSpec((1, 1), lambda i: (0, 0)),
        ],
        out_shape=[
            jax.ShapeDtypeStruct((nb, kb), jnp.float32),
            jax.ShapeDtypeStruct((1, d_dim), jnp.float32),
            jax.ShapeDtypeStruct((1, 1), jnp.float32),
        ],
        scratch_shapes=[
            pltpu.VMEM((s_rows, 128), jnp.float32),
            pltpu.VMEM((1, s_rows), jnp.float32),
            pltpu.VMEM((nb, s_rows), jnp.float32),
            pltpu.VMEM((4, 128), jnp.float32),
            pltpu.SMEM((2,), jnp.float32),
        ],
    )(mu, sig, x)

    quantised = y.reshape(d_dim)
    loss_s = loss.reshape(())
    return (quantised, loss_s, loss_s, dists2d.reshape(k_dim))
